# baseline (speedup 1.0000x reference)
import functools

import jax
from jax import lax
import jax.numpy as jnp
from jax.experimental import pallas as pl
from jax.experimental.pallas import tpu as pltpu
from jax.experimental.pallas import tpu_sc as plsc

B, S, D, H, E, K = 2, 2048, 1024, 16, 8, 2
DH = D // H
N = B * S
EPS = 1e-5


HPG = 2
HD2 = HPG * DH
QC = 512


def _attn_kernel(x_ref, wq_ref, wk_ref, wv_ref, bq_ref, o_ref):
    x_bf = x_ref[0].astype(jnp.bfloat16)
    wq = wq_ref[0].astype(jnp.bfloat16)
    wk = wk_ref[0].astype(jnp.bfloat16)
    wv = wv_ref[0].astype(jnp.bfloat16)
    q2 = jnp.dot(x_bf, wq, preferred_element_type=jnp.float32)
    q2 = (q2 + bq_ref[0]) * (1.0 / jnp.sqrt(jnp.float32(DH)))
    k2 = jnp.dot(x_bf, wk, preferred_element_type=jnp.float32)
    v2 = jnp.dot(x_bf, wv, preferred_element_type=jnp.float32)
    q2 = q2.astype(jnp.bfloat16)
    k2 = k2.astype(jnp.bfloat16)
    v2 = v2.astype(jnp.bfloat16)
    for hh in range(HPG):
        k_h = k2[:, hh * DH:(hh + 1) * DH]
        v_h = v2[:, hh * DH:(hh + 1) * DH]
        for c in range(S // QC):
            q_h = q2[c * QC:(c + 1) * QC, hh * DH:(hh + 1) * DH]
            scores = jax.lax.dot_general(
                q_h, k_h, (((1,), (1,)), ((), ())),
                preferred_element_type=jnp.float32)
            m = jnp.max(scores, axis=1, keepdims=True)
            p = jnp.exp(scores - m)
            attn = (p / jnp.sum(p, axis=1, keepdims=True)).astype(jnp.bfloat16)
            o = jnp.dot(attn, v_h, preferred_element_type=jnp.float32)
            o_ref[0, c * QC:(c + 1) * QC, hh * DH:(hh + 1) * DH] = o


def _run_attention(x, wq_r, wk_r, wv_r, bq):
    return pl.pallas_call(
        _attn_kernel,
        grid=(B, H // HPG),
        in_specs=[
            pl.BlockSpec((1, S, D), lambda b, g: (b, 0, 0)),
            pl.BlockSpec((1, D, HD2), lambda b, g: (g, 0, 0)),
            pl.BlockSpec((1, D, HD2), lambda b, g: (g, 0, 0)),
            pl.BlockSpec((1, D, HD2), lambda b, g: (g, 0, 0)),
            pl.BlockSpec((1, 1, HD2), lambda b, g: (g, 0, 0)),
        ],
        out_specs=pl.BlockSpec((1, S, HD2), lambda b, g: (b, 0, g)),
        out_shape=jax.ShapeDtypeStruct((B, S, D), jnp.float32),
        compiler_params=pltpu.CompilerParams(
            dimension_semantics=("arbitrary", "arbitrary")),
    )(x, wq_r, wk_r, wv_r, bq)


TB2 = 512


def _mid_kernel(o_ref, x_ref, wo_ref, beff_ref, g1_ref, b1_ref,
                gw_ref, gb_ref, h_ref, gate_ref, o1_ref, o2_ref,
                w0_ref, w1_ref):
    o_bf = o_ref[...].astype(jnp.bfloat16)
    wo = wo_ref[...].astype(jnp.bfloat16)
    ao = jnp.dot(o_bf, wo, preferred_element_type=jnp.float32) + beff_ref[0]
    r = x_ref[...] + ao
    mu = jnp.mean(r, axis=1, keepdims=True)
    c = r - mu
    var = jnp.mean(c * c, axis=1, keepdims=True)
    h = c / jnp.sqrt(var + EPS) * g1_ref[0] + b1_ref[0]
    h_ref[...] = h
    logits = jax.lax.dot_general(
        h.astype(jnp.bfloat16), gw_ref[...].astype(jnp.bfloat16),
        (((1,), (0,)), ((), ())),
        preferred_element_type=jnp.float32) + gb_ref[0]
    lm = jnp.max(logits, axis=1, keepdims=True)
    pe = jnp.exp(logits - lm)
    gate = pe / jnp.sum(pe, axis=1, keepdims=True)
    gate_ref[...] = gate
    iot = jax.lax.broadcasted_iota(jnp.int32, (TB2, E), 1)
    i1 = jnp.argmax(gate, axis=1)
    m1 = jnp.max(gate, axis=1)
    mask1 = iot == i1[:, None]
    g2 = jnp.where(mask1, -1.0, gate)
    i2 = jnp.argmax(g2, axis=1)
    m2 = jnp.max(g2, axis=1)
    mask2 = iot == i2[:, None]
    ws = m1 + m2
    o1_ref[...] = mask1.astype(jnp.float32)
    o2_ref[...] = mask2.astype(jnp.float32)
    w0_ref[...] = (m1 / ws)[:, None]
    w1_ref[...] = (m2 / ws)[:, None]


def _run_mid(o2, x2, wo_t, b_eff, ln1_g, ln1_b, gate_w, gate_b):
    return pl.pallas_call(
        _mid_kernel,
        grid=(N // TB2,),
        in_specs=[
            pl.BlockSpec((TB2, D), lambda i: (i, 0)),
            pl.BlockSpec((TB2, D), lambda i: (i, 0)),
            pl.BlockSpec((D, D), lambda i: (0, 0)),
            pl.BlockSpec((1, D), lambda i: (0, 0)),
            pl.BlockSpec((1, D), lambda i: (0, 0)),
            pl.BlockSpec((1, D), lambda i: (0, 0)),
            pl.BlockSpec((D, E), lambda i: (0, 0)),
            pl.BlockSpec((1, E), lambda i: (0, 0)),
        ],
        out_specs=[
            pl.BlockSpec((TB2, D), lambda i: (i, 0)),
            pl.BlockSpec((TB2, E), lambda i: (i, 0)),
            pl.BlockSpec((TB2, E), lambda i: (i, 0)),
            pl.BlockSpec((TB2, E), lambda i: (i, 0)),
            pl.BlockSpec((TB2, 1), lambda i: (i, 0)),
            pl.BlockSpec((TB2, 1), lambda i: (i, 0)),
        ],
        out_shape=[
            jax.ShapeDtypeStruct((N, D), jnp.float32),
            jax.ShapeDtypeStruct((N, E), jnp.float32),
            jax.ShapeDtypeStruct((N, E), jnp.float32),
            jax.ShapeDtypeStruct((N, E), jnp.float32),
            jax.ShapeDtypeStruct((N, 1), jnp.float32),
            jax.ShapeDtypeStruct((N, 1), jnp.float32),
        ],
        compiler_params=pltpu.CompilerParams(
            dimension_semantics=("arbitrary",)),
    )(o2, x2, wo_t, b_eff, ln1_g, ln1_b, gate_w, gate_b)


RB = 512
NA = 2 * N
CAP = N


def _route_kernel(ot_ref, slots_ref, counts_ref, carry_ref):
    i = pl.program_id(0)

    @pl.when(i == 0)
    def _():
        carry_ref[...] = jnp.zeros_like(carry_ref)

    ot = ot_ref[...]
    r = jax.lax.broadcasted_iota(jnp.int32, (RB, RB), 0)
    c = jax.lax.broadcasted_iota(jnp.int32, (RB, RB), 1)
    m = (r < c).astype(jnp.float32)
    prefix = jax.lax.dot_general(
        ot, m, (((1,), (0,)), ((), ())),
        preferred_element_type=jnp.float32) + carry_ref[:, 0:1]
    rank = jnp.sum(ot * prefix, axis=0, keepdims=True)
    evec = jnp.sum(
        ot * jax.lax.broadcasted_iota(jnp.int32, (E, RB), 0).astype(
            jnp.float32),
        axis=0, keepdims=True)
    slots_ref[0] = (evec * CAP + rank).astype(jnp.int32)
    carry_ref[...] += jnp.broadcast_to(
        jnp.sum(ot, axis=1, keepdims=True), carry_ref.shape)

    @pl.when(i == NA // RB - 1)
    def _():
        counts_ref[...] = carry_ref[:, 0:1]


def _run_route(o_t):
    return pl.pallas_call(
        _route_kernel,
        grid=(NA // RB,),
        in_specs=[pl.BlockSpec((E, RB), lambda i: (0, i))],
        out_specs=[
            pl.BlockSpec((1, 1, RB), lambda i: (i, 0, 0)),
            pl.BlockSpec((E, 1), lambda i: (0, 0)),
        ],
        out_shape=[
            jax.ShapeDtypeStruct((NA // RB, 1, RB), jnp.int32),
            jax.ShapeDtypeStruct((E, 1), jnp.float32),
        ],
        scratch_shapes=[pltpu.VMEM((E, 128), jnp.float32)],
        compiler_params=pltpu.CompilerParams(
            dimension_semantics=("arbitrary",)),
    )(o_t)


NC, NS = 2, 16
NW = NC * NS
TPW = N // NW
CHUNK = 64
@functools.lru_cache(maxsize=None)
def _sc_mesh():
    return plsc.VectorSubcoreMesh(core_axis_name="c", subcore_axis_name="s")


def _sc_dispatch_kernel(h_hbm, s0_hbm, s1_hbm, hd_hbm, idx_v, rows_v, sem):
    wid = lax.axis_index("s") * NC + lax.axis_index("c")
    base = wid * TPW
    for cnk in range(TPW // CHUNK):
        off = base + cnk * CHUNK
        pltpu.sync_copy(h_hbm.at[pl.ds(off, CHUNK)], rows_v)
        pltpu.sync_copy(s0_hbm.at[pl.ds(off, CHUNK)], idx_v)
        pltpu.async_copy(rows_v, hd_hbm.at[idx_v], sem).wait()
        pltpu.sync_copy(s1_hbm.at[pl.ds(off, CHUNK)], idx_v)
        pltpu.async_copy(rows_v, hd_hbm.at[idx_v], sem).wait()


def _run_sc_dispatch(h, slots0, slots1):
    return pl.kernel(
        _sc_dispatch_kernel,
        mesh=_sc_mesh(),
        out_type=jax.ShapeDtypeStruct((E * CAP, D), jnp.float32),
        scratch_types=[
            pltpu.VMEM((CHUNK,), jnp.int32),
            pltpu.VMEM((CHUNK, D), jnp.float32),
            pltpu.SemaphoreType.DMA,
        ],
    )(h, slots0, slots1)


def _sc_combine_kernel(y_hbm, s0_hbm, s1_hbm, yb0_hbm, yb1_hbm,
                       idx_v, rows_v, sem):
    wid = lax.axis_index("s") * NC + lax.axis_index("c")
    base = wid * TPW
    for cnk in range(TPW // CHUNK):
        off = base + cnk * CHUNK
        pltpu.sync_copy(s0_hbm.at[pl.ds(off, CHUNK)], idx_v)
        pltpu.async_copy(y_hbm.at[idx_v], rows_v, sem).wait()
        pltpu.sync_copy(rows_v, yb0_hbm.at[pl.ds(off, CHUNK)])
        pltpu.sync_copy(s1_hbm.at[pl.ds(off, CHUNK)], idx_v)
        pltpu.async_copy(y_hbm.at[idx_v], rows_v, sem).wait()
        pltpu.sync_copy(rows_v, yb1_hbm.at[pl.ds(off, CHUNK)])


def _run_sc_combine(y_disp, slots0, slots1):
    return pl.kernel(
        _sc_combine_kernel,
        mesh=_sc_mesh(),
        out_type=[
            jax.ShapeDtypeStruct((N, D), jnp.float32),
            jax.ShapeDtypeStruct((N, D), jnp.float32),
        ],
        scratch_types=[
            pltpu.VMEM((CHUNK,), jnp.int32),
            pltpu.VMEM((CHUNK, D), jnp.float32),
            pltpu.SemaphoreType.DMA,
        ],
    )(y_disp, slots0, slots1)


CBLK = 256
CB = CAP // CBLK


def _gmm_kernel(counts_ref, h_ref, ew_ref, eb_ref, y_ref):
    e = pl.program_id(0)
    cb = pl.program_id(1)

    @pl.when(cb * CBLK < counts_ref[e])
    def _():
        h_bf = h_ref[...].astype(jnp.bfloat16)
        ew = ew_ref[0].astype(jnp.bfloat16)
        y_ref[...] = jnp.dot(
            h_bf, ew, preferred_element_type=jnp.float32) + eb_ref[0]


def _run_gmm(counts, h_disp, expert_w, expert_b3):
    grid_spec = pltpu.PrefetchScalarGridSpec(
        num_scalar_prefetch=1,
        grid=(E, CB),
        in_specs=[
            pl.BlockSpec((CBLK, D), lambda e, cb, cnt: (e * CB + cb, 0)),
            pl.BlockSpec((1, D, D), lambda e, cb, cnt: (e, 0, 0)),
            pl.BlockSpec((1, 1, D), lambda e, cb, cnt: (e, 0, 0)),
        ],
        out_specs=pl.BlockSpec((CBLK, D), lambda e, cb, cnt: (e * CB + cb, 0)),
    )
    return pl.pallas_call(
        _gmm_kernel,
        grid_spec=grid_spec,
        out_shape=jax.ShapeDtypeStruct((E * CAP, D), jnp.float32),
        compiler_params=pltpu.CompilerParams(
            dimension_semantics=("arbitrary", "arbitrary")),
    )(counts, h_disp, expert_w, expert_b3)


TB4 = 512


def _final_kernel(h_ref, y0_ref, y1_ref, w0_ref, w1_ref, g2_ref, b2_ref,
                  out_ref):
    r = h_ref[...] + w0_ref[...] * y0_ref[...] + w1_ref[...] * y1_ref[...]
    mu = jnp.mean(r, axis=1, keepdims=True)
    c = r - mu
    var = jnp.mean(c * c, axis=1, keepdims=True)
    out_ref[...] = c / jnp.sqrt(var + EPS) * g2_ref[0] + b2_ref[0]


def _run_final(h, yb0, yb1, w0, w1, ln2_g, ln2_b):
    return pl.pallas_call(
        _final_kernel,
        grid=(N // TB4,),
        in_specs=[
            pl.BlockSpec((TB4, D), lambda i: (i, 0)),
            pl.BlockSpec((TB4, D), lambda i: (i, 0)),
            pl.BlockSpec((TB4, D), lambda i: (i, 0)),
            pl.BlockSpec((TB4, 1), lambda i: (i, 0)),
            pl.BlockSpec((TB4, 1), lambda i: (i, 0)),
            pl.BlockSpec((1, D), lambda i: (0, 0)),
            pl.BlockSpec((1, D), lambda i: (0, 0)),
        ],
        out_specs=pl.BlockSpec((TB4, D), lambda i: (i, 0)),
        out_shape=jax.ShapeDtypeStruct((N, D), jnp.float32),
        compiler_params=pltpu.CompilerParams(
            dimension_semantics=("arbitrary",)),
    )(h, yb0, yb1, w0, w1, ln2_g, ln2_b)


@jax.jit
def kernel(x, in_proj_w, in_proj_b, out_proj_w, out_proj_b, ln1_g, ln1_b,
           ln2_g, ln2_b, gate_w, gate_b, expert_w, expert_b):
    wq_r = in_proj_w[:D].T.reshape(D, H // HPG, HD2).transpose(1, 0, 2)
    wk_r = in_proj_w[D:2 * D].T.reshape(D, H // HPG, HD2).transpose(1, 0, 2)
    wv_r = in_proj_w[2 * D:].T.reshape(D, H // HPG, HD2).transpose(1, 0, 2)
    bq = in_proj_b[:D].reshape(H // HPG, 1, HD2)
    bv = in_proj_b[2 * D:]
    b_eff = (out_proj_b + bv @ out_proj_w.T).reshape(1, D)

    o = _run_attention(x, wq_r, wk_r, wv_r, bq)

    o2 = o.reshape(N, D)
    x2 = x.reshape(N, D)
    h, gate, o1, o2m, w0, w1 = _run_mid(
        o2, x2, out_proj_w.T, b_eff, ln1_g.reshape(1, D), ln1_b.reshape(1, D),
        gate_w, gate_b.reshape(1, E))

    o_t = jnp.concatenate([o1, o2m], axis=0).T
    slots3, counts_f = _run_route(o_t)
    slots = slots3.reshape(NA)
    slots0, slots1 = slots[:N], slots[N:]
    counts = counts_f.reshape(E).astype(jnp.int32)

    h_disp = _run_sc_dispatch(h, slots0, slots1)
    y_disp = _run_gmm(counts, h_disp, expert_w, expert_b.reshape(E, 1, D))
    yb0, yb1 = _run_sc_combine(y_disp, slots0, slots1)

    out = _run_final(h, yb0, yb1, w0, w1,
                     ln2_g.reshape(1, D), ln2_b.reshape(1, D))
    return out.reshape(B, S, D), gate.reshape(B, S, E)


# speedup vs baseline: 1.1679x; 1.1679x over previous
import functools

import jax
from jax import lax
import jax.numpy as jnp
from jax.experimental import pallas as pl
from jax.experimental.pallas import tpu as pltpu
from jax.experimental.pallas import tpu_sc as plsc

B, S, D, H, E, K = 2, 2048, 1024, 16, 8, 2
DH = D // H
N = B * S
EPS = 1e-5


TB0 = 1024


def _qkv_kernel(x_ref, w_ref, b_ref, qkv_ref):
    x_bf = x_ref[...].astype(jnp.bfloat16)
    y = jnp.dot(x_bf, w_ref[...], preferred_element_type=jnp.float32)
    qkv_ref[...] = (y + b_ref[0]).astype(jnp.bfloat16)


def _run_qkv(x2, w_qkv_bf, b_qkv):
    return pl.pallas_call(
        _qkv_kernel,
        grid=(N // TB0,),
        in_specs=[
            pl.BlockSpec((TB0, D), lambda i: (i, 0)),
            pl.BlockSpec((D, 3 * D), lambda i: (0, 0)),
            pl.BlockSpec((1, 3 * D), lambda i: (0, 0)),
        ],
        out_specs=pl.BlockSpec((TB0, 3 * D), lambda i: (i, 0)),
        out_shape=jax.ShapeDtypeStruct((N, 3 * D), jnp.bfloat16),
        compiler_params=pltpu.CompilerParams(
            dimension_semantics=("arbitrary",)),
    )(x2, w_qkv_bf, b_qkv)


HPG = 2
HD2 = HPG * DH
QC = 1024
NG = H // HPG


def _attn_kernel(q_ref, k_ref, v_ref, o_ref):
    q2 = q_ref[0]
    k2 = k_ref[0]
    v2 = v_ref[0]
    for hh in range(HPG):
        k_h = k2[:, hh * DH:(hh + 1) * DH]
        v_h = v2[:, hh * DH:(hh + 1) * DH]
        for c in range(S // QC):
            q_h = q2[c * QC:(c + 1) * QC, hh * DH:(hh + 1) * DH]
            scores = jax.lax.dot_general(
                q_h, k_h, (((1,), (1,)), ((), ())),
                preferred_element_type=jnp.float32)
            m = jnp.max(scores, axis=1, keepdims=True)
            p = jnp.exp(scores - m)
            attn = (p / jnp.sum(p, axis=1, keepdims=True)).astype(jnp.bfloat16)
            o = jnp.dot(attn, v_h, preferred_element_type=jnp.float32)
            o_ref[0, c * QC:(c + 1) * QC, hh * DH:(hh + 1) * DH] = o


def _run_attention(qkv3):
    return pl.pallas_call(
        _attn_kernel,
        grid=(B, NG),
        in_specs=[
            pl.BlockSpec((1, S, HD2), lambda b, g: (b, 0, g)),
            pl.BlockSpec((1, S, HD2), lambda b, g: (b, 0, NG + g)),
            pl.BlockSpec((1, S, HD2), lambda b, g: (b, 0, 2 * NG + g)),
        ],
        out_specs=pl.BlockSpec((1, S, HD2), lambda b, g: (b, 0, g)),
        out_shape=jax.ShapeDtypeStruct((B, S, D), jnp.float32),
        compiler_params=pltpu.CompilerParams(
            dimension_semantics=("arbitrary", "arbitrary")),
    )(qkv3, qkv3, qkv3)


TB2 = 512


def _mid_kernel(o_ref, x_ref, wo_ref, beff_ref, g1_ref, b1_ref,
                gw_ref, gb_ref, h_ref, gate_ref, o1_ref, o2_ref,
                w0_ref, w1_ref):
    o_bf = o_ref[...].astype(jnp.bfloat16)
    wo = wo_ref[...].astype(jnp.bfloat16)
    ao = jnp.dot(o_bf, wo, preferred_element_type=jnp.float32) + beff_ref[0]
    r = x_ref[...] + ao
    mu = jnp.mean(r, axis=1, keepdims=True)
    c = r - mu
    var = jnp.mean(c * c, axis=1, keepdims=True)
    h = c / jnp.sqrt(var + EPS) * g1_ref[0] + b1_ref[0]
    h_ref[...] = h
    logits = jax.lax.dot_general(
        h.astype(jnp.bfloat16), gw_ref[...].astype(jnp.bfloat16),
        (((1,), (0,)), ((), ())),
        preferred_element_type=jnp.float32) + gb_ref[0]
    lm = jnp.max(logits, axis=1, keepdims=True)
    pe = jnp.exp(logits - lm)
    gate = pe / jnp.sum(pe, axis=1, keepdims=True)
    gate_ref[...] = gate
    iot = jax.lax.broadcasted_iota(jnp.int32, (TB2, E), 1)
    i1 = jnp.argmax(gate, axis=1)
    m1 = jnp.max(gate, axis=1)
    mask1 = iot == i1[:, None]
    g2 = jnp.where(mask1, -1.0, gate)
    i2 = jnp.argmax(g2, axis=1)
    m2 = jnp.max(g2, axis=1)
    mask2 = iot == i2[:, None]
    ws = m1 + m2
    o1_ref[...] = mask1.astype(jnp.float32)
    o2_ref[...] = mask2.astype(jnp.float32)
    w0_ref[...] = (m1 / ws)[:, None]
    w1_ref[...] = (m2 / ws)[:, None]


def _run_mid(o2, x2, wo_t, b_eff, ln1_g, ln1_b, gate_w, gate_b):
    return pl.pallas_call(
        _mid_kernel,
        grid=(N // TB2,),
        in_specs=[
            pl.BlockSpec((TB2, D), lambda i: (i, 0)),
            pl.BlockSpec((TB2, D), lambda i: (i, 0)),
            pl.BlockSpec((D, D), lambda i: (0, 0)),
            pl.BlockSpec((1, D), lambda i: (0, 0)),
            pl.BlockSpec((1, D), lambda i: (0, 0)),
            pl.BlockSpec((1, D), lambda i: (0, 0)),
            pl.BlockSpec((D, E), lambda i: (0, 0)),
            pl.BlockSpec((1, E), lambda i: (0, 0)),
        ],
        out_specs=[
            pl.BlockSpec((TB2, D), lambda i: (i, 0)),
            pl.BlockSpec((TB2, E), lambda i: (i, 0)),
            pl.BlockSpec((TB2, E), lambda i: (i, 0)),
            pl.BlockSpec((TB2, E), lambda i: (i, 0)),
            pl.BlockSpec((TB2, 1), lambda i: (i, 0)),
            pl.BlockSpec((TB2, 1), lambda i: (i, 0)),
        ],
        out_shape=[
            jax.ShapeDtypeStruct((N, D), jnp.float32),
            jax.ShapeDtypeStruct((N, E), jnp.float32),
            jax.ShapeDtypeStruct((N, E), jnp.float32),
            jax.ShapeDtypeStruct((N, E), jnp.float32),
            jax.ShapeDtypeStruct((N, 1), jnp.float32),
            jax.ShapeDtypeStruct((N, 1), jnp.float32),
        ],
        compiler_params=pltpu.CompilerParams(
            dimension_semantics=("arbitrary",)),
    )(o2, x2, wo_t, b_eff, ln1_g, ln1_b, gate_w, gate_b)


RB = 512
NA = 2 * N
CAP = N


def _route_kernel(ot_ref, slots_ref, counts_ref, carry_ref):
    i = pl.program_id(0)

    @pl.when(i == 0)
    def _():
        carry_ref[...] = jnp.zeros_like(carry_ref)

    ot = ot_ref[...]
    r = jax.lax.broadcasted_iota(jnp.int32, (RB, RB), 0)
    c = jax.lax.broadcasted_iota(jnp.int32, (RB, RB), 1)
    m = (r < c).astype(jnp.float32)
    prefix = jax.lax.dot_general(
        ot, m, (((1,), (0,)), ((), ())),
        preferred_element_type=jnp.float32) + carry_ref[:, 0:1]
    rank = jnp.sum(ot * prefix, axis=0, keepdims=True)
    evec = jnp.sum(
        ot * jax.lax.broadcasted_iota(jnp.int32, (E, RB), 0).astype(
            jnp.float32),
        axis=0, keepdims=True)
    slots_ref[0] = (evec * CAP + rank).astype(jnp.int32)
    carry_ref[...] += jnp.broadcast_to(
        jnp.sum(ot, axis=1, keepdims=True), carry_ref.shape)

    @pl.when(i == NA // RB - 1)
    def _():
        counts_ref[...] = carry_ref[:, 0:1]


def _run_route(o_t):
    return pl.pallas_call(
        _route_kernel,
        grid=(NA // RB,),
        in_specs=[pl.BlockSpec((E, RB), lambda i: (0, i))],
        out_specs=[
            pl.BlockSpec((1, 1, RB), lambda i: (i, 0, 0)),
            pl.BlockSpec((E, 1), lambda i: (0, 0)),
        ],
        out_shape=[
            jax.ShapeDtypeStruct((NA // RB, 1, RB), jnp.int32),
            jax.ShapeDtypeStruct((E, 1), jnp.float32),
        ],
        scratch_shapes=[pltpu.VMEM((E, 128), jnp.float32)],
        compiler_params=pltpu.CompilerParams(
            dimension_semantics=("arbitrary",)),
    )(o_t)


NC, NS = 2, 16
NW = NC * NS
TPW = N // NW
CHUNK = 64
@functools.lru_cache(maxsize=None)
def _sc_mesh():
    return plsc.VectorSubcoreMesh(core_axis_name="c", subcore_axis_name="s")


def _sc_dispatch_kernel(h_hbm, s0_hbm, s1_hbm, hd_hbm, idx_v, rows_v, sem):
    wid = lax.axis_index("s") * NC + lax.axis_index("c")
    base = wid * TPW
    for cnk in range(TPW // CHUNK):
        off = base + cnk * CHUNK
        pltpu.sync_copy(h_hbm.at[pl.ds(off, CHUNK)], rows_v)
        pltpu.sync_copy(s0_hbm.at[pl.ds(off, CHUNK)], idx_v)
        pltpu.async_copy(rows_v, hd_hbm.at[idx_v], sem).wait()
        pltpu.sync_copy(s1_hbm.at[pl.ds(off, CHUNK)], idx_v)
        pltpu.async_copy(rows_v, hd_hbm.at[idx_v], sem).wait()


def _run_sc_dispatch(h, slots0, slots1):
    return pl.kernel(
        _sc_dispatch_kernel,
        mesh=_sc_mesh(),
        out_type=jax.ShapeDtypeStruct((E * CAP, D), jnp.float32),
        scratch_types=[
            pltpu.VMEM((CHUNK,), jnp.int32),
            pltpu.VMEM((CHUNK, D), jnp.float32),
            pltpu.SemaphoreType.DMA,
        ],
    )(h, slots0, slots1)


def _sc_combine_kernel(y_hbm, s0_hbm, s1_hbm, yb0_hbm, yb1_hbm,
                       idx_v, rows_v, sem):
    wid = lax.axis_index("s") * NC + lax.axis_index("c")
    base = wid * TPW
    for cnk in range(TPW // CHUNK):
        off = base + cnk * CHUNK
        pltpu.sync_copy(s0_hbm.at[pl.ds(off, CHUNK)], idx_v)
        pltpu.async_copy(y_hbm.at[idx_v], rows_v, sem).wait()
        pltpu.sync_copy(rows_v, yb0_hbm.at[pl.ds(off, CHUNK)])
        pltpu.sync_copy(s1_hbm.at[pl.ds(off, CHUNK)], idx_v)
        pltpu.async_copy(y_hbm.at[idx_v], rows_v, sem).wait()
        pltpu.sync_copy(rows_v, yb1_hbm.at[pl.ds(off, CHUNK)])


def _run_sc_combine(y_disp, slots0, slots1):
    return pl.kernel(
        _sc_combine_kernel,
        mesh=_sc_mesh(),
        out_type=[
            jax.ShapeDtypeStruct((N, D), jnp.float32),
            jax.ShapeDtypeStruct((N, D), jnp.float32),
        ],
        scratch_types=[
            pltpu.VMEM((CHUNK,), jnp.int32),
            pltpu.VMEM((CHUNK, D), jnp.float32),
            pltpu.SemaphoreType.DMA,
        ],
    )(y_disp, slots0, slots1)


CBLK = 256
CB = CAP // CBLK


def _gmm_kernel(counts_ref, h_ref, ew_ref, eb_ref, y_ref):
    e = pl.program_id(0)
    cb = pl.program_id(1)

    @pl.when(cb * CBLK < counts_ref[e])
    def _():
        h_bf = h_ref[...].astype(jnp.bfloat16)
        ew = ew_ref[0].astype(jnp.bfloat16)
        y_ref[...] = jnp.dot(
            h_bf, ew, preferred_element_type=jnp.float32) + eb_ref[0]


def _run_gmm(counts, h_disp, expert_w, expert_b3):
    def in_map(e, cb, cnt):
        return (jnp.where(cb * CBLK < cnt[e], e * CB + cb, 0), 0)

    def out_map(e, cb, cnt):
        return (jnp.where(cb * CBLK < cnt[e], e * CB + cb, E * CB), 0)

    grid_spec = pltpu.PrefetchScalarGridSpec(
        num_scalar_prefetch=1,
        grid=(E, CB),
        in_specs=[
            pl.BlockSpec((CBLK, D), in_map),
            pl.BlockSpec((1, D, D), lambda e, cb, cnt: (e, 0, 0)),
            pl.BlockSpec((1, 1, D), lambda e, cb, cnt: (e, 0, 0)),
        ],
        out_specs=pl.BlockSpec((CBLK, D), out_map),
    )
    y_ext = pl.pallas_call(
        _gmm_kernel,
        grid_spec=grid_spec,
        out_shape=jax.ShapeDtypeStruct((E * CAP + CBLK, D), jnp.float32),
        compiler_params=pltpu.CompilerParams(
            dimension_semantics=("arbitrary", "arbitrary")),
    )(counts, h_disp, expert_w, expert_b3)
    return y_ext


TB4 = 512


def _final_kernel(h_ref, y0_ref, y1_ref, w0_ref, w1_ref, g2_ref, b2_ref,
                  out_ref):
    r = h_ref[...] + w0_ref[...] * y0_ref[...] + w1_ref[...] * y1_ref[...]
    mu = jnp.mean(r, axis=1, keepdims=True)
    c = r - mu
    var = jnp.mean(c * c, axis=1, keepdims=True)
    out_ref[...] = c / jnp.sqrt(var + EPS) * g2_ref[0] + b2_ref[0]


def _run_final(h, yb0, yb1, w0, w1, ln2_g, ln2_b):
    return pl.pallas_call(
        _final_kernel,
        grid=(N // TB4,),
        in_specs=[
            pl.BlockSpec((TB4, D), lambda i: (i, 0)),
            pl.BlockSpec((TB4, D), lambda i: (i, 0)),
            pl.BlockSpec((TB4, D), lambda i: (i, 0)),
            pl.BlockSpec((TB4, 1), lambda i: (i, 0)),
            pl.BlockSpec((TB4, 1), lambda i: (i, 0)),
            pl.BlockSpec((1, D), lambda i: (0, 0)),
            pl.BlockSpec((1, D), lambda i: (0, 0)),
        ],
        out_specs=pl.BlockSpec((TB4, D), lambda i: (i, 0)),
        out_shape=jax.ShapeDtypeStruct((N, D), jnp.float32),
        compiler_params=pltpu.CompilerParams(
            dimension_semantics=("arbitrary",)),
    )(h, yb0, yb1, w0, w1, ln2_g, ln2_b)


@jax.jit
def kernel(x, in_proj_w, in_proj_b, out_proj_w, out_proj_b, ln1_g, ln1_b,
           ln2_g, ln2_b, gate_w, gate_b, expert_w, expert_b):
    scale = 1.0 / jnp.sqrt(jnp.float32(DH))
    w_qkv = jnp.concatenate(
        [in_proj_w[:D].T * scale, in_proj_w[D:2 * D].T, in_proj_w[2 * D:].T],
        axis=1).astype(jnp.bfloat16)
    b_qkv = jnp.concatenate(
        [in_proj_b[:D] * scale, jnp.zeros((2 * D,), jnp.float32)]
    ).reshape(1, 3 * D)
    bv = in_proj_b[2 * D:]
    b_eff = (out_proj_b + bv @ out_proj_w.T).reshape(1, D)

    x2 = x.reshape(N, D)
    qkv = _run_qkv(x2, w_qkv, b_qkv)
    o = _run_attention(qkv.reshape(B, S, 3 * D))

    o2 = o.reshape(N, D)
    h, gate, o1, o2m, w0, w1 = _run_mid(
        o2, x2, out_proj_w.T, b_eff, ln1_g.reshape(1, D), ln1_b.reshape(1, D),
        gate_w, gate_b.reshape(1, E))

    o_t = jnp.concatenate([o1, o2m], axis=0).T
    slots3, counts_f = _run_route(o_t)
    slots = slots3.reshape(NA)
    slots0, slots1 = slots[:N], slots[N:]
    counts = counts_f.reshape(E).astype(jnp.int32)

    h_disp = _run_sc_dispatch(h, slots0, slots1)
    y_disp = _run_gmm(counts, h_disp, expert_w, expert_b.reshape(E, 1, D))
    yb0, yb1 = _run_sc_combine(y_disp, slots0, slots1)

    out = _run_final(h, yb0, yb1, w0, w1,
                     ln2_g.reshape(1, D), ln2_b.reshape(1, D))
    return out.reshape(B, S, D), gate.reshape(B, S, E)
